# baseline (speedup 1.0000x reference)
import functools

import jax
import jax.numpy as jnp
from jax import lax
from jax.experimental import pallas as pl
from jax.experimental.pallas import tpu as pltpu
from jax.experimental.pallas import tpu_sc as plsc

_VOCAB = 100000
_DIM = 128
_B, _S = 4, 4096
_NUM_LAYERS = 12

_NTOT = _B * _S
_NC, _NS = 2, 16
_NW = _NC * _NS
_PER_W = _NTOT // _NW
_CHUNK = 128
_ROWS_W = _PER_W // _CHUNK
_WBUF = 256
_NBUF = 3
_NGIF = 1


def _gather3(idx2d, w0, w1, w2):
    mesh = plsc.VectorSubcoreMesh(core_axis_name="c", subcore_axis_name="s")

    @functools.partial(
        pl.kernel,
        mesh=mesh,
        out_type=[jax.ShapeDtypeStruct((_NTOT, _DIM), jnp.float32)] * _NUM_LAYERS,
        scratch_types=[
            pltpu.VMEM((_PER_W,), jnp.int32),
            pltpu.VMEM((_NBUF, _WBUF, _DIM), jnp.float32),
            pltpu.SemaphoreType.DMA((_NBUF,)),
            pltpu.SemaphoreType.DMA((_NBUF,)),
        ],
    )
    def body(idx_hbm, t0, t1, t2, *rest):
        outs = rest[:_NUM_LAYERS]
        idx_v, bufs, gsem, wsem = rest[_NUM_LAYERS:]
        wid = lax.axis_index("s") * _NC + lax.axis_index("c")
        pltpu.sync_copy(
            idx_hbm.at[wid // (_S // _PER_W), pl.ds((wid % (_S // _PER_W)) * _PER_W, _PER_W)],
            idx_v)

        tabs = (t0, t1, t2)
        cpb = _WBUF // _CHUNK
        tasks = [(t, b) for t in range(3) for b in range(_ROWS_W // cpb)]
        n = len(tasks)

        def issue_gathers(k):
            t, b = tasks[k]
            slot = k % _NBUF
            return [
                pltpu.async_copy(
                    tabs[t].at[idx_v.at[pl.ds((b * cpb + h) * _CHUNK, _CHUNK)]],
                    bufs.at[slot].at[pl.ds(h * _CHUNK, _CHUNK)],
                    gsem.at[slot])
                for h in range(cpb)
            ]

        def issue_writebacks(k):
            t, b = tasks[k]
            slot = k % _NBUF
            row0 = wid * _PER_W + b * _WBUF
            return [
                pltpu.async_copy(
                    bufs.at[slot], outs[l].at[pl.ds(row0, _WBUF)],
                    wsem.at[slot])
                for l in range(t, _NUM_LAYERS, 3)
            ]

        gh = [None] * n
        wh = [None] * n
        for k in range(_NGIF):
            gh[k] = issue_gathers(k)
        for k in range(n):
            for h in gh[k]:
                h.wait()
            wh[k] = issue_writebacks(k)
            nxt = k + _NGIF
            if nxt < n:
                if nxt >= _NBUF:
                    for h in wh[nxt - _NBUF]:
                        h.wait()
                gh[nxt] = issue_gathers(nxt)
        for k in range(n - _NBUF, n):
            for h in wh[k]:
                h.wait()

    return body(idx2d, w0, w1, w2)


def kernel(input_seq, W0, W1, W2):
    outs = _gather3(input_seq, W0, W1, W2)
    return tuple(o.reshape(_B, _S, _DIM) for o in outs)


# speedup vs baseline: 1.0116x; 1.0116x over previous
import functools

import jax
import jax.numpy as jnp
from jax import lax
from jax.experimental import pallas as pl
from jax.experimental.pallas import tpu as pltpu
from jax.experimental.pallas import tpu_sc as plsc

_VOCAB = 100000
_DIM = 128
_B, _S = 4, 4096
_NUM_LAYERS = 12

_NTOT = _B * _S
_NC, _NS = 2, 16
_NW = _NC * _NS
_PER_W = _NTOT // _NW
_CHUNK = 128
_ROWS_W = _PER_W // _CHUNK
_WBUF = 256
_NBUF = 3
_NGIF = 2


def _gather3(idx2d, w0, w1, w2):
    mesh = plsc.VectorSubcoreMesh(core_axis_name="c", subcore_axis_name="s")

    @functools.partial(
        pl.kernel,
        mesh=mesh,
        out_type=[jax.ShapeDtypeStruct((_NTOT, _DIM), jnp.float32)] * _NUM_LAYERS,
        scratch_types=[
            pltpu.VMEM((_PER_W,), jnp.int32),
            pltpu.VMEM((_NBUF, _WBUF, _DIM), jnp.float32),
            pltpu.SemaphoreType.DMA((_NBUF,)),
            pltpu.SemaphoreType.DMA((_NBUF,)),
        ],
    )
    def body(idx_hbm, t0, t1, t2, *rest):
        outs = rest[:_NUM_LAYERS]
        idx_v, bufs, gsem, wsem = rest[_NUM_LAYERS:]
        wid = lax.axis_index("s") * _NC + lax.axis_index("c")
        pltpu.sync_copy(
            idx_hbm.at[wid // (_S // _PER_W), pl.ds((wid % (_S // _PER_W)) * _PER_W, _PER_W)],
            idx_v)

        tabs = (t0, t1, t2)
        cpb = _WBUF // _CHUNK
        tasks = [(t, b) for t in range(3) for b in range(_ROWS_W // cpb)]
        n = len(tasks)

        def issue_gathers(k):
            t, b = tasks[k]
            slot = k % _NBUF
            return [
                pltpu.async_copy(
                    tabs[t].at[idx_v.at[pl.ds((b * cpb + h) * _CHUNK, _CHUNK)]],
                    bufs.at[slot].at[pl.ds(h * _CHUNK, _CHUNK)],
                    gsem.at[slot])
                for h in range(cpb)
            ]

        def issue_writebacks(k):
            t, b = tasks[k]
            slot = k % _NBUF
            row0 = wid * _PER_W + b * _WBUF
            return [
                pltpu.async_copy(
                    bufs.at[slot], outs[l].at[pl.ds(row0, _WBUF)],
                    wsem.at[slot])
                for l in range(t, _NUM_LAYERS, 3)
            ]

        gh = [None] * n
        wh = [None] * n
        for k in range(_NGIF):
            gh[k] = issue_gathers(k)
        for k in range(n):
            for h in gh[k]:
                h.wait()
            wh[k] = issue_writebacks(k)
            nxt = k + _NGIF
            if nxt < n:
                if nxt >= _NBUF:
                    for h in wh[nxt - _NBUF]:
                        h.wait()
                gh[nxt] = issue_gathers(nxt)
        for k in range(n - _NBUF, n):
            for h in wh[k]:
                h.wait()

    return body(idx2d, w0, w1, w2)


def kernel(input_seq, W0, W1, W2):
    outs = _gather3(input_seq, W0, W1, W2)
    return tuple(o.reshape(_B, _S, _DIM) for o in outs)


# speedup vs baseline: 1.0150x; 1.0034x over previous
import functools

import jax
import jax.numpy as jnp
from jax import lax
from jax.experimental import pallas as pl
from jax.experimental.pallas import tpu as pltpu
from jax.experimental.pallas import tpu_sc as plsc

_VOCAB = 100000
_DIM = 128
_B, _S = 4, 4096
_NUM_LAYERS = 12

_NTOT = _B * _S
_NC, _NS = 2, 16
_NW = _NC * _NS
_PER_W = _NTOT // _NW
_CHUNK = 128
_ROWS_W = _PER_W // _CHUNK
_WBUF = 256
_NBUF = 3
_NGIF = 2


def _gather3(idx2d, w0, w1, w2):
    mesh = plsc.VectorSubcoreMesh(core_axis_name="c", subcore_axis_name="s")

    @functools.partial(
        pl.kernel,
        mesh=mesh,
        out_type=[jax.ShapeDtypeStruct((_NTOT, _DIM), jnp.float32)] * _NUM_LAYERS,
        scratch_types=[
            pltpu.VMEM((_PER_W,), jnp.int32),
            pltpu.VMEM((_NBUF, _WBUF, _DIM), jnp.float32),
            pltpu.SemaphoreType.DMA((_NBUF,)),
            pltpu.SemaphoreType.DMA((_NBUF,)),
        ],
    )
    def body(idx_hbm, t0, t1, t2, *rest):
        outs = rest[:_NUM_LAYERS]
        idx_v, bufs, gsem, wsem = rest[_NUM_LAYERS:]
        wid = lax.axis_index("c") * _NS + lax.axis_index("s")
        pltpu.sync_copy(
            idx_hbm.at[wid // (_S // _PER_W), pl.ds((wid % (_S // _PER_W)) * _PER_W, _PER_W)],
            idx_v)

        tabs = (t0, t1, t2)
        cpb = _WBUF // _CHUNK
        tasks = [(t, b) for t in range(3) for b in range(_ROWS_W // cpb)]
        n = len(tasks)

        def issue_gathers(k):
            t, b = tasks[k]
            slot = k % _NBUF
            return [
                pltpu.async_copy(
                    tabs[t].at[idx_v.at[pl.ds((b * cpb + h) * _CHUNK, _CHUNK)]],
                    bufs.at[slot].at[pl.ds(h * _CHUNK, _CHUNK)],
                    gsem.at[slot])
                for h in range(cpb)
            ]

        def issue_writebacks(k):
            t, b = tasks[k]
            slot = k % _NBUF
            row0 = wid * _PER_W + b * _WBUF
            return [
                pltpu.async_copy(
                    bufs.at[slot], outs[l].at[pl.ds(row0, _WBUF)],
                    wsem.at[slot])
                for l in range(t, _NUM_LAYERS, 3)
            ]

        gh = [None] * n
        wh = [None] * n
        for k in range(_NGIF):
            gh[k] = issue_gathers(k)
        for k in range(n):
            for h in gh[k]:
                h.wait()
            wh[k] = issue_writebacks(k)
            nxt = k + _NGIF
            if nxt < n:
                if nxt >= _NBUF:
                    for h in wh[nxt - _NBUF]:
                        h.wait()
                gh[nxt] = issue_gathers(nxt)
        for k in range(n - _NBUF, n):
            for h in wh[k]:
                h.wait()

    return body(idx2d, w0, w1, w2)


def kernel(input_seq, W0, W1, W2):
    outs = _gather3(input_seq, W0, W1, W2)
    return tuple(o.reshape(_B, _S, _DIM) for o in outs)


# speedup vs baseline: 1.0197x; 1.0046x over previous
import functools

import jax
import jax.numpy as jnp
from jax import lax
from jax.experimental import pallas as pl
from jax.experimental.pallas import tpu as pltpu
from jax.experimental.pallas import tpu_sc as plsc

_VOCAB = 100000
_DIM = 128
_B, _S = 4, 4096
_NUM_LAYERS = 12

_NTOT = _B * _S
_NC, _NS = 2, 16
_NW = _NC * _NS
_PER_W = _NTOT // _NW
_CHUNK = 128
_ROWS_W = _PER_W // _CHUNK
_WBUF = 256
_NBUF = 3
_NGIF = 2


def _gather3(idx2d, w0, w1, w2):
    mesh = plsc.VectorSubcoreMesh(core_axis_name="c", subcore_axis_name="s")

    @functools.partial(
        pl.kernel,
        mesh=mesh,
        out_type=[jax.ShapeDtypeStruct((_NTOT, _DIM), jnp.float32)] * _NUM_LAYERS,
        scratch_types=[
            pltpu.VMEM((_PER_W,), jnp.int32),
            pltpu.VMEM((_NBUF, _WBUF, _DIM), jnp.float32),
            pltpu.SemaphoreType.DMA((_NBUF,)),
            pltpu.SemaphoreType.DMA((_NBUF,)),
        ],
    )
    def body(idx_hbm, t0, t1, t2, *rest):
        outs = rest[:_NUM_LAYERS]
        idx_v, bufs, gsem, wsem = rest[_NUM_LAYERS:]
        wid = lax.axis_index("c") * _NS + lax.axis_index("s")
        pltpu.sync_copy(
            idx_hbm.at[wid // (_S // _PER_W), pl.ds((wid % (_S // _PER_W)) * _PER_W, _PER_W)],
            idx_v)

        tabs = (t0, t1, t2)
        cpb = _WBUF // _CHUNK
        tasks = [(t, b) for b in range(_ROWS_W // cpb) for t in range(3)]
        n = len(tasks)

        def issue_gathers(k):
            t, b = tasks[k]
            slot = k % _NBUF
            return [
                pltpu.async_copy(
                    tabs[t].at[idx_v.at[pl.ds((b * cpb + h) * _CHUNK, _CHUNK)]],
                    bufs.at[slot].at[pl.ds(h * _CHUNK, _CHUNK)],
                    gsem.at[slot])
                for h in range(cpb)
            ]

        def issue_writebacks(k):
            t, b = tasks[k]
            slot = k % _NBUF
            row0 = wid * _PER_W + b * _WBUF
            return [
                pltpu.async_copy(
                    bufs.at[slot], outs[l].at[pl.ds(row0, _WBUF)],
                    wsem.at[slot])
                for l in range(t, _NUM_LAYERS, 3)
            ]

        gh = [None] * n
        wh = [None] * n
        for k in range(_NGIF):
            gh[k] = issue_gathers(k)
        for k in range(n):
            for h in gh[k]:
                h.wait()
            wh[k] = issue_writebacks(k)
            nxt = k + _NGIF
            if nxt < n:
                if nxt >= _NBUF:
                    for h in wh[nxt - _NBUF]:
                        h.wait()
                gh[nxt] = issue_gathers(nxt)
        for k in range(n - _NBUF, n):
            for h in wh[k]:
                h.wait()

    return body(idx2d, w0, w1, w2)


def kernel(input_seq, W0, W1, W2):
    outs = _gather3(input_seq, W0, W1, W2)
    return tuple(o.reshape(_B, _S, _DIM) for o in outs)
